# baseline (speedup 1.0000x reference)
import functools

import jax
import jax.numpy as jnp
from jax import lax
from jax.experimental import pallas as pl
from jax.experimental.pallas import tpu as pltpu
from jax.experimental.pallas import tpu_sc as plsc

_MAXLEN = 2048
_EMBED = 64
_BATCH = 64
_NW = 32
_ROWS = _BATCH * _MAXLEN
_RPW = _ROWS // _NW
_CHUNK = 512
_NCHUNK = _RPW // _CHUNK
_SUB = 128
_NSUB = _CHUNK // _SUB
_LANES = 16


@functools.partial(
    pl.kernel,
    mesh=plsc.VectorSubcoreMesh(core_axis_name="c", subcore_axis_name="s"),
    out_type=jax.ShapeDtypeStruct((_ROWS * _EMBED,), jnp.float32),
    scratch_types=[
        pltpu.VMEM((_CHUNK,), jnp.int32),
        pltpu.VMEM((_CHUNK, 2 * _EMBED), jnp.float32),
        pltpu.VMEM((_CHUNK * _EMBED,), jnp.float32),
        pltpu.SemaphoreType.DMA,
    ],
)
def _tokpos(x_hbm, tok_hbm, pos_hbm, out_hbm, idx_v, tok_v, acc_v, sem):
    wid = lax.axis_index("s") * 2 + lax.axis_index("c")
    base = wid * _RPW
    for c in range(_NCHUNK):
        gbase = base + c * _CHUNK
        pbase = ((c * _CHUNK) % _MAXLEN) * _EMBED
        pltpu.sync_copy(x_hbm.at[pl.ds(gbase, _CHUNK)], idx_v)
        pltpu.sync_copy(pos_hbm.at[pl.ds(pbase, _CHUNK * _EMBED)], acc_v)
        copies = [
            pltpu.async_copy(
                tok_hbm.at[idx_v.at[pl.ds(k * _SUB, _SUB)]],
                tok_v.at[pl.ds(k * _SUB, _SUB)],
                sem,
            )
            for k in range(_NSUB)
        ]
        for cp in copies:
            cp.wait()

        def body(r, carry):
            for e in range(_EMBED // _LANES):
                sl = pl.ds(r * _EMBED + e * _LANES, _LANES)
                acc_v[sl] = acc_v[sl] + tok_v[r, pl.ds(e * _LANES, _LANES)]
            return carry

        lax.fori_loop(0, _CHUNK, body, 0)
        pltpu.sync_copy(acc_v, out_hbm.at[pl.ds(gbase * _EMBED, _CHUNK * _EMBED)])


def kernel(x, token_table, pos_table):
    xf = x.reshape(-1).astype(jnp.int32)
    t128 = jnp.pad(token_table, ((0, 0), (0, 2 * _EMBED - _EMBED)))
    posf = pos_table.reshape(-1)
    out = _tokpos(xf, t128, posf)
    return out.reshape(x.shape[0], x.shape[1], _EMBED)


# speedup vs baseline: 1.4108x; 1.4108x over previous
import functools

import jax
import jax.numpy as jnp
from jax import lax
from jax.experimental import pallas as pl
from jax.experimental.pallas import tpu as pltpu
from jax.experimental.pallas import tpu_sc as plsc

_MAXLEN = 2048
_EMBED = 64
_BATCH = 64
_NW = 32
_ROWS = _BATCH * _MAXLEN
_RPW = _ROWS // _NW
_PPW = _RPW // _BATCH
_CHUNK = 512
_NCHUNK = _RPW // _CHUNK
_PPC = _CHUNK // _BATCH
_SUB = 128
_NSUB = _CHUNK // _SUB
_LANES = 16


@functools.partial(
    pl.kernel,
    mesh=plsc.VectorSubcoreMesh(core_axis_name="c", subcore_axis_name="s"),
    out_type=jax.ShapeDtypeStruct((_ROWS, _EMBED), jnp.float32),
    scratch_types=[
        pltpu.VMEM((_CHUNK,), jnp.int32),
        pltpu.VMEM((_CHUNK, _EMBED), jnp.float32),
        pltpu.VMEM((_PPW, _EMBED), jnp.float32),
        pltpu.VMEM((_NSUB, _SUB), jnp.int32),
        pltpu.SemaphoreType.DMA,
        pltpu.SemaphoreType.DMA,
    ],
    compiler_params=pltpu.CompilerParams(use_tc_tiling_on_sc=False),
)
def _tokpos(xt_hbm, tok_hbm, pos_hbm, out_hbm, idx_v, tok_v, pos_v, oidx_v,
            gsem, ssem):
    wid = lax.axis_index("s") * 2 + lax.axis_index("c")
    base = wid * _RPW
    pbase = wid * _PPW
    pltpu.sync_copy(pos_hbm.at[pl.ds(pbase, _PPW)], pos_v)
    iota_b = lax.iota(jnp.int32, _LANES) * _MAXLEN

    for c in range(_NCHUNK):
        gbase = base + c * _CHUNK
        pltpu.sync_copy(xt_hbm.at[pl.ds(gbase, _CHUNK)], idx_v)
        gathers = [
            pltpu.async_copy(
                tok_hbm.at[idx_v.at[pl.ds(k * _SUB, _SUB)]],
                tok_v.at[pl.ds(k * _SUB, _SUB)],
                gsem,
            )
            for k in range(_NSUB)
        ]
        for k in range(_NSUB):
            for h in range(_SUB // _LANES):
                p_abs = pbase + c * _PPC + 2 * k + h // (_BATCH // _LANES)
                b_off = (h % (_BATCH // _LANES)) * _LANES * _MAXLEN
                oidx_v[k, pl.ds(h * _LANES, _LANES)] = iota_b + (b_off + p_abs)
        for cp in gathers:
            cp.wait()

        for q in range(_PPC):
            row0 = q * _BATCH
            pos_regs = [pos_v[c * _PPC + q, pl.ds(e * _LANES, _LANES)]
                        for e in range(_EMBED // _LANES)]

            def body(r, regs):
                for e in range(_EMBED // _LANES):
                    sl = pl.ds(e * _LANES, _LANES)
                    tok_v[row0 + r, sl] = tok_v[row0 + r, sl] + regs[e]
                return regs

            lax.fori_loop(0, _BATCH, body, tuple(pos_regs))

        scatters = [
            pltpu.async_copy(
                tok_v.at[pl.ds(k * _SUB, _SUB)],
                out_hbm.at[oidx_v.at[k]],
                ssem,
            )
            for k in range(_NSUB)
        ]
        for cp in scatters:
            cp.wait()


def kernel(x, token_table, pos_table):
    xt = x.T.reshape(-1).astype(jnp.int32)
    out = _tokpos(xt, token_table, pos_table)
    return out.reshape(x.shape[0], x.shape[1], _EMBED)
